# baseline (speedup 1.0000x reference)
import functools

import jax
import jax.numpy as jnp
from jax import lax
from jax.experimental import pallas as pl
from jax.experimental.pallas import tpu as pltpu
from jax.experimental.pallas import tpu_sc as plsc

_N = 10000
_E = 320000
_CH = 128
_NW = 32
_F32 = jnp.float32

_EP = 327680
_NP = 10240
_ROWS = 1024
_NCHUNK = _EP // _CH
_BASE = _NCHUNK // _NW
_RPT = _NP // 16


def _mesh():
    return plsc.VectorSubcoreMesh(core_axis_name="c", subcore_axis_name="s")


def _make_sc_agg(H):
    hb = _BASE // 2

    @functools.partial(
        pl.kernel,
        mesh=_mesh(),
        out_type=jax.ShapeDtypeStruct((2, _NP, H), _F32),
        scratch_types=[
            pltpu.VMEM((hb, _CH), jnp.int32),
            pltpu.VMEM((hb, _CH), jnp.int32),
            pltpu.VMEM((_CH, H), _F32),
            pltpu.VMEM((_CH, H), _F32),
            pltpu.VMEM_SHARED((_NP, H), _F32),
            pltpu.SemaphoreType.DMA,
            pltpu.SemaphoreType.DMA,
        ],
    )
    def agg(hl_hbm, src_hbm, dst_hbm, z_hbm, out_hbm,
            srcs_v, dsts_v, rows0, rows1, acc_sh, sem0, sem1):
        c = lax.axis_index("c")
        s = lax.axis_index("s")
        w = s * 2 + c

        pltpu.sync_copy(z_hbm.at[pl.ds(s * _RPT, _RPT)],
                        acc_sh.at[pl.ds(s * _RPT, _RPT)])

        def body2(j, carry):
            i0 = 2 * j
            pltpu.make_async_copy(hl_hbm.at[srcs_v.at[i0]], rows0,
                                  sem0).wait()
            pltpu.sync_copy(rows0, acc_sh.at[dsts_v.at[i0]], add=True)

            @pl.when(j < hb // 2 - 1)
            def _():
                pltpu.async_copy(hl_hbm.at[srcs_v.at[i0 + 2]], rows0, sem0)

            pltpu.make_async_copy(hl_hbm.at[srcs_v.at[i0 + 1]], rows1,
                                  sem1).wait()
            pltpu.sync_copy(rows1, acc_sh.at[dsts_v.at[i0 + 1]], add=True)

            @pl.when(j < hb // 2 - 1)
            def _():
                pltpu.async_copy(hl_hbm.at[srcs_v.at[i0 + 3]], rows1, sem1)

            return carry

        for half in range(2):
            base_row = w * _BASE + half * hb
            pltpu.sync_copy(src_hbm.at[pl.ds(base_row, hb)], srcs_v)
            pltpu.sync_copy(dst_hbm.at[pl.ds(base_row, hb)], dsts_v)
            pltpu.async_copy(hl_hbm.at[srcs_v.at[0]], rows0, sem0)
            pltpu.async_copy(hl_hbm.at[srcs_v.at[1]], rows1, sem1)
            if half == 0:
                plsc.subcore_barrier()
            lax.fori_loop(0, hb // 2, body2, 0)

        plsc.subcore_barrier()

        pltpu.sync_copy(acc_sh.at[pl.ds(s * _RPT, _RPT)],
                        out_hbm.at[c, pl.ds(s * _RPT, _RPT)])

    return agg


def _make_sc_deg():
    @functools.partial(
        pl.kernel,
        mesh=_mesh(),
        out_type=jax.ShapeDtypeStruct((2, _NP), _F32),
        scratch_types=[
            pltpu.VMEM((_BASE, _CH), jnp.int32),
            pltpu.VMEM((_CH,), _F32),
            pltpu.VMEM_SHARED((_NP,), _F32),
        ],
    )
    def deg(dst_hbm, zn_hbm, out_hbm, dsts_v, ones_v, deg_sh):
        c = lax.axis_index("c")
        s = lax.axis_index("s")
        w = s * 2 + c

        for j in range(_CH // 16):
            ones_v[pl.ds(j * 16, 16)] = jnp.full((16,), 1.0, dtype=_F32)

        @pl.when(s == 0)
        def _():
            pltpu.sync_copy(zn_hbm, deg_sh)

        pltpu.sync_copy(dst_hbm.at[pl.ds(w * _BASE, _BASE)], dsts_v)

        plsc.subcore_barrier()

        def chunk(i, carry):
            pltpu.sync_copy(ones_v, deg_sh.at[dsts_v.at[i]], add=True)
            return carry

        lax.fori_loop(0, _BASE, chunk, 0)

        plsc.subcore_barrier()

        @pl.when(s == 0)
        def _():
            pltpu.sync_copy(deg_sh, out_hbm.at[c])

    return deg


def _mm_body(x_ref, w_ref, o_ref):
    o_ref[...] = jnp.dot(x_ref[...], w_ref[...],
                         preferred_element_type=_F32)


def _tc_mm(x, w):
    n, d = x.shape
    h = w.shape[1]
    return pl.pallas_call(
        _mm_body,
        grid=(n // _ROWS,),
        in_specs=[pl.BlockSpec((_ROWS, d), lambda i: (i, 0)),
                  pl.BlockSpec((d, h), lambda i: (0, 0))],
        out_specs=pl.BlockSpec((_ROWS, h), lambda i: (i, 0)),
        out_shape=jax.ShapeDtypeStruct((n, h), _F32),
    )(x, w)


def _mid1_body(s_ref, dg_ref, x_ref, wr_ref, b_ref, wl_ref,
               h_ref, hl_ref, inv_ref):
    deg = dg_ref[0] + dg_ref[1]
    inv = 1.0 / jnp.maximum(deg, 1.0)
    agg = (s_ref[0] + s_ref[1]) * inv
    h = jnp.maximum(
        agg + jnp.dot(x_ref[...], wr_ref[...], preferred_element_type=_F32)
        + b_ref[...], 0.0)
    h_ref[...] = h
    hl_ref[...] = jnp.dot(h, wl_ref[...], preferred_element_type=_F32)
    inv_ref[...] = inv


def _tc_mid1(S, dg, x, wr, b, wl):
    n, d = x.shape
    h2 = wl.shape[1]
    return pl.pallas_call(
        _mid1_body,
        grid=(n // _ROWS,),
        in_specs=[pl.BlockSpec((2, _ROWS, d), lambda i: (0, i, 0)),
                  pl.BlockSpec((2, _ROWS, 1), lambda i: (0, i, 0)),
                  pl.BlockSpec((_ROWS, d), lambda i: (i, 0)),
                  pl.BlockSpec((d, d), lambda i: (0, 0)),
                  pl.BlockSpec((1, d), lambda i: (0, 0)),
                  pl.BlockSpec((d, h2), lambda i: (0, 0))],
        out_specs=[pl.BlockSpec((_ROWS, d), lambda i: (i, 0)),
                   pl.BlockSpec((_ROWS, h2), lambda i: (i, 0)),
                   pl.BlockSpec((_ROWS, 1), lambda i: (i, 0))],
        out_shape=[jax.ShapeDtypeStruct((n, d), _F32),
                   jax.ShapeDtypeStruct((n, h2), _F32),
                   jax.ShapeDtypeStruct((n, 1), _F32)],
    )(S, dg, x, wr, b, wl)


def _mid2_body(s_ref, inv_ref, x_ref, wr_ref, b_ref, h_ref):
    agg = (s_ref[0] + s_ref[1]) * inv_ref[...]
    h_ref[...] = jnp.maximum(
        agg + jnp.dot(x_ref[...], wr_ref[...], preferred_element_type=_F32)
        + b_ref[...], 0.0)


def _tc_mid2(S, inv, x, wr, b):
    n, d = x.shape
    return pl.pallas_call(
        _mid2_body,
        grid=(n // _ROWS,),
        in_specs=[pl.BlockSpec((2, _ROWS, d), lambda i: (0, i, 0)),
                  pl.BlockSpec((_ROWS, 1), lambda i: (i, 0)),
                  pl.BlockSpec((_ROWS, d), lambda i: (i, 0)),
                  pl.BlockSpec((d, d), lambda i: (0, 0)),
                  pl.BlockSpec((1, d), lambda i: (0, 0))],
        out_specs=pl.BlockSpec((_ROWS, d), lambda i: (i, 0)),
        out_shape=jax.ShapeDtypeStruct((n, d), _F32),
    )(S, inv, x, wr, b)


def _fin_body(s_ref, inv_ref, x_ref, wl_ref, wr_ref, b_ref, o_ref):
    agg = (s_ref[0] + s_ref[1]) * inv_ref[...]
    o_ref[...] = jnp.maximum(
        jnp.dot(agg, wl_ref[...], preferred_element_type=_F32)
        + jnp.dot(x_ref[...], wr_ref[...], preferred_element_type=_F32)
        + b_ref[...], 0.0)


def _tc_fin(S, inv, x, wl, wr, b):
    n, d = x.shape
    h = wr.shape[1]
    return pl.pallas_call(
        _fin_body,
        grid=(n // _ROWS,),
        in_specs=[pl.BlockSpec((2, _ROWS, d), lambda i: (0, i, 0)),
                  pl.BlockSpec((_ROWS, 1), lambda i: (i, 0)),
                  pl.BlockSpec((_ROWS, d), lambda i: (i, 0)),
                  pl.BlockSpec((d, h), lambda i: (0, 0)),
                  pl.BlockSpec((d, h), lambda i: (0, 0)),
                  pl.BlockSpec((1, h), lambda i: (0, 0))],
        out_specs=pl.BlockSpec((_ROWS, h), lambda i: (i, 0)),
        out_shape=jax.ShapeDtypeStruct((n, h), _F32),
    )(S, inv, x, wl, wr, b)


_sc_agg128 = _make_sc_agg(128)
_sc_deg = _make_sc_deg()


def kernel(x, edge_index, Wl1, Wr1, b1, Wl2, Wr2, b2, Wl3, Wr3, b3):
    pad_e = _EP - _E
    src = jnp.concatenate(
        [edge_index[0], jnp.zeros((pad_e,), jnp.int32)]).reshape(_NCHUNK, _CH)
    dst = jnp.concatenate(
        [edge_index[1], jnp.full((pad_e,), _N, jnp.int32)]).reshape(_NCHUNK, _CH)
    xp = jnp.concatenate([x, jnp.zeros((_NP - _N, 128), _F32)])
    z128 = jnp.zeros((_NP, 128), _F32)
    zn = jnp.zeros((_NP,), _F32)

    dg = _sc_deg(dst, zn)
    hl1 = _tc_mm(xp, Wl1)
    S1 = _sc_agg128(hl1, src, dst, z128)
    h2, hl2, inv = _tc_mid1(S1, dg.reshape(2, _NP, 1), xp,
                            Wr1, b1.reshape(1, 128), Wl2)
    S2 = _sc_agg128(hl2, src, dst, z128)
    h3 = _tc_mid2(S2, inv, h2, Wr2, b2.reshape(1, 128))
    S3 = _sc_agg128(h3, src, dst, z128)
    out = _tc_fin(S3, inv, h3, Wl3, Wr3, b3.reshape(1, 64))
    return out[:_N]
